# baseline (speedup 1.0000x reference)
import functools

import jax
import jax.numpy as jnp
from jax.experimental import pallas as pl
from jax.experimental.pallas import tpu as pltpu


def _rowscale_kernel(h_ref, w_ref, o_ref):
    h = h_ref[...]
    w = w_ref[...]
    topk = w.shape[1]
    acc = h * w[:, 0:1]
    for k in range(1, topk):
        acc = acc + h * w[:, k : k + 1]
    o_ref[...] = acc


@functools.partial(jax.jit, static_argnames=())
def kernel(hidden_states, routing_indices, routing_weights):
    del routing_indices
    num_tokens, hidden_dim = hidden_states.shape
    topk = routing_weights.shape[1]
    w = routing_weights.astype(hidden_states.dtype)

    blk = 256
    while num_tokens % blk != 0:
        blk //= 2
    grid = (num_tokens // blk,)

    return pl.pallas_call(
        _rowscale_kernel,
        grid=grid,
        in_specs=[
            pl.BlockSpec((blk, hidden_dim), lambda i: (i, 0)),
            pl.BlockSpec((blk, topk), lambda i: (i, 0)),
        ],
        out_specs=pl.BlockSpec((blk, hidden_dim), lambda i: (i, 0)),
        out_shape=jax.ShapeDtypeStruct((num_tokens, hidden_dim), hidden_states.dtype),
        compiler_params=pltpu.CompilerParams(
            dimension_semantics=("arbitrary",),
        ),
    )(hidden_states, w)


# speedup vs baseline: 1.1399x; 1.1399x over previous
import functools

import jax
import jax.numpy as jnp
from jax.experimental import pallas as pl
from jax.experimental.pallas import tpu as pltpu


def _rowscale_kernel(h_ref, w_ref, o_ref):
    h = h_ref[...]
    w = w_ref[...]
    topk = w.shape[1]
    acc = h * w[:, 0:1]
    for k in range(1, topk):
        acc = acc + h * w[:, k : k + 1]
    o_ref[...] = acc


@functools.partial(jax.jit, static_argnames=())
def kernel(hidden_states, routing_indices, routing_weights):
    del routing_indices
    num_tokens, hidden_dim = hidden_states.shape
    topk = routing_weights.shape[1]
    w = routing_weights.astype(hidden_states.dtype)

    blk = 1024
    while num_tokens % blk != 0:
        blk //= 2
    grid = (num_tokens // blk,)

    return pl.pallas_call(
        _rowscale_kernel,
        grid=grid,
        in_specs=[
            pl.BlockSpec((blk, hidden_dim), lambda i: (i, 0)),
            pl.BlockSpec((blk, topk), lambda i: (i, 0)),
        ],
        out_specs=pl.BlockSpec((blk, hidden_dim), lambda i: (i, 0)),
        out_shape=jax.ShapeDtypeStruct((num_tokens, hidden_dim), hidden_states.dtype),
        compiler_params=pltpu.CompilerParams(
            dimension_semantics=("arbitrary",),
        ),
    )(hidden_states, w)


# speedup vs baseline: 1.1446x; 1.0041x over previous
import functools

import jax
import jax.numpy as jnp
from jax.experimental import pallas as pl
from jax.experimental.pallas import tpu as pltpu


def _rowscale_kernel(h_ref, w_ref, o_ref):
    h = h_ref[...]
    w = w_ref[...]
    topk = w.shape[1]
    acc = h * w[:, 0:1]
    for k in range(1, topk):
        acc = acc + h * w[:, k : k + 1]
    o_ref[...] = acc


@functools.partial(jax.jit, static_argnames=())
def kernel(hidden_states, routing_indices, routing_weights):
    del routing_indices
    num_tokens, hidden_dim = hidden_states.shape
    topk = routing_weights.shape[1]
    w = routing_weights

    blk = 1024
    while num_tokens % blk != 0:
        blk //= 2
    grid = (num_tokens // blk,)

    return pl.pallas_call(
        _rowscale_kernel,
        grid=grid,
        in_specs=[
            pl.BlockSpec((blk, hidden_dim), lambda i: (i, 0)),
            pl.BlockSpec((blk, topk), lambda i: (i, 0)),
        ],
        out_specs=pl.BlockSpec((blk, hidden_dim), lambda i: (i, 0)),
        out_shape=jax.ShapeDtypeStruct((num_tokens, hidden_dim), hidden_states.dtype),
        compiler_params=pltpu.CompilerParams(
            dimension_semantics=("parallel",),
        ),
    )(hidden_states, w)


# speedup vs baseline: 1.1612x; 1.0145x over previous
import jax
import jax.numpy as jnp
from jax.experimental import pallas as pl
from jax.experimental.pallas import tpu as pltpu

NBUF = 3
BLK = 1024


def _pipelined_kernel(h_hbm, w_vmem, o_hbm, inbuf, outbuf, scale_buf, in_sems, out_sems):
    num_tokens = h_hbm.shape[0]
    nblocks = num_tokens // BLK

    w = w_vmem[...]
    scale_buf[...] = jnp.sum(w, axis=1, keepdims=True)

    def in_copy(t, slot):
        return pltpu.make_async_copy(
            h_hbm.at[pl.ds(t * BLK, BLK), :], inbuf.at[slot], in_sems.at[slot]
        )

    def out_copy(t, slot):
        return pltpu.make_async_copy(
            outbuf.at[slot], o_hbm.at[pl.ds(t * BLK, BLK), :], out_sems.at[slot]
        )

    for s in range(NBUF):
        in_copy(s, s).start()

    def body(t, _):
        slot = jax.lax.rem(t, NBUF)
        in_copy(t, slot).wait()

        @pl.when(t >= NBUF)
        def _():
            out_copy(t - NBUF, slot).wait()

        s = scale_buf[pl.ds(t * BLK, BLK), :]
        outbuf[slot] = inbuf[slot] * s
        out_copy(t, slot).start()

        @pl.when(t + NBUF < nblocks)
        def _():
            in_copy(t + NBUF, slot).start()

        return 0

    jax.lax.fori_loop(0, nblocks, body, 0)

    for s in range(NBUF):
        t = nblocks - NBUF + s
        out_copy(t, jax.lax.rem(jnp.int32(t), NBUF)).wait()


def kernel(hidden_states, routing_indices, routing_weights):
    del routing_indices
    num_tokens, hidden_dim = hidden_states.shape
    topk = routing_weights.shape[1]

    return pl.pallas_call(
        _pipelined_kernel,
        in_specs=[
            pl.BlockSpec(memory_space=pltpu.MemorySpace.HBM),
            pl.BlockSpec(memory_space=pltpu.VMEM),
        ],
        out_specs=pl.BlockSpec(memory_space=pltpu.MemorySpace.HBM),
        out_shape=jax.ShapeDtypeStruct((num_tokens, hidden_dim), hidden_states.dtype),
        scratch_shapes=[
            pltpu.VMEM((NBUF, BLK, hidden_dim), hidden_states.dtype),
            pltpu.VMEM((NBUF, BLK, hidden_dim), hidden_states.dtype),
            pltpu.VMEM((num_tokens, 1), jnp.float32),
            pltpu.SemaphoreType.DMA((NBUF,)),
            pltpu.SemaphoreType.DMA((NBUF,)),
        ],
    )(hidden_states, routing_weights)
